# speedup vs baseline: 1.6593x; 1.6593x over previous
import functools

import jax
import jax.numpy as jnp
from jax import lax
from jax.experimental import pallas as pl
from jax.experimental.pallas import tpu as pltpu
from jax.experimental.pallas import tpu_sc as plsc

BATCH = 4
SEQ = 2048
EMBED_DIM = 128

NUM_CORES = 2
NUM_SUBCORES = 16
LANES = 16
NW = NUM_CORES * NUM_SUBCORES

TOKENS = BATCH * SEQ
CHUNK = TOKENS // NW
IDX_MINOR = 128
K = CHUNK // IDX_MINOR

_mesh = plsc.VectorSubcoreMesh(core_axis_name="c", subcore_axis_name="s")


@functools.partial(
    pl.kernel,
    mesh=_mesh,
    out_type=jax.ShapeDtypeStruct((TOKENS, EMBED_DIM), jnp.float32),
    scratch_types=[
        pltpu.VMEM((K, IDX_MINOR), jnp.int32),
        pltpu.VMEM((K, IDX_MINOR), jnp.int32),
        pltpu.VMEM((CHUNK, EMBED_DIM), jnp.float32),
        pltpu.VMEM((CHUNK, EMBED_DIM), jnp.float32),
        pltpu.SemaphoreType.DMA,
        pltpu.SemaphoreType.DMA,
    ],
)
def _embed_sc(x_hbm, xp_hbm, tok_hbm, pos_hbm, out_hbm,
              xi, pi, tok_v, pos_v, sem_t, sem_p):
    wid = lax.axis_index("s") * NUM_CORES + lax.axis_index("c")
    base = wid * CHUNK

    pltpu.sync_copy(x_hbm.at[wid], xi)
    pltpu.sync_copy(xp_hbm.at[wid], pi)

    copies = []
    for k in range(K):
        copies.append(pltpu.async_copy(
            tok_hbm.at[xi.at[k]], tok_v.at[pl.ds(k * IDX_MINOR, IDX_MINOR)],
            sem_t))
        copies.append(pltpu.async_copy(
            pos_hbm.at[pi.at[k]], pos_v.at[pl.ds(k * IDX_MINOR, IDX_MINOR)],
            sem_p))
    for c in copies:
        c.wait()

    def body(i, _):
        for j in range(EMBED_DIM // LANES):
            sl = pl.ds(j * LANES, LANES)
            tok_v[i, sl] = tok_v[i, sl] + pos_v[i, sl]
        return _

    lax.fori_loop(0, CHUNK, body, 0)

    pltpu.sync_copy(tok_v, out_hbm.at[pl.ds(base, CHUNK)])


def kernel(x, x_pos, tok_embed, pos_embed):
    xf = x.reshape(NW, K, IDX_MINOR).astype(jnp.int32)
    xpf = x_pos.reshape(NW, K, IDX_MINOR).astype(jnp.int32)
    out = _embed_sc(xf, xpf, tok_embed, pos_embed)
    return out.reshape(BATCH, SEQ, EMBED_DIM)
